# Initial kernel scaffold; baseline (speedup 1.0000x reference)
import jax
import jax.numpy as jnp
from jax.experimental import pallas as pl


def kernel(item_inputs, table):
    raise NotImplementedError("write your pallas kernel here")


# speedup vs baseline: 1.0946x; 1.0946x over previous
import functools

import jax
import jax.numpy as jnp
from jax import lax
from jax.experimental import pallas as pl
from jax.experimental.pallas import tpu as pltpu
from jax.experimental.pallas import tpu_sc as plsc

BATCH = 16384
HIST = 50
EMBED_DIM = 32
TOTAL = BATCH * HIST

NUM_CORES = 2
NUM_SUBCORES = 16
NUM_WORKERS = NUM_CORES * NUM_SUBCORES
PER_WORKER = TOTAL // NUM_WORKERS

DMA_CHUNK = 128
GROUP = 1024
DMAS_PER_GROUP = GROUP // DMA_CHUNK
GROUPS_PER_WORKER = PER_WORKER // GROUP
IDX_ROWS_PER_GROUP = GROUP // DMA_CHUNK


def _gather_body(idx_hbm, table_hbm, out_hbm, idx_v, rows_v, sem):
    wid = lax.axis_index("s") * NUM_CORES + lax.axis_index("c")
    base = wid * PER_WORKER

    def group_fn(g, carry):
        gbase = base + g * GROUP
        idx_row = pl.multiple_of(gbase // DMA_CHUNK, 8)
        pltpu.sync_copy(idx_hbm.at[pl.ds(idx_row, IDX_ROWS_PER_GROUP)], idx_v)
        copies = []
        for j in range(DMAS_PER_GROUP):
            copies.append(
                pltpu.async_copy(
                    table_hbm.at[idx_v.at[j]],
                    rows_v.at[pl.ds(j * DMA_CHUNK, DMA_CHUNK)],
                    sem,
                )
            )
        for c in copies:
            c.wait()
        pltpu.sync_copy(rows_v, out_hbm.at[pl.ds(gbase, GROUP)])
        return carry

    lax.fori_loop(0, GROUPS_PER_WORKER, group_fn, 0)


@functools.partial(jax.jit, donate_argnums=())
def _sc_gather(idx2d, table):
    mesh = plsc.VectorSubcoreMesh(core_axis_name="c", subcore_axis_name="s")
    run = pl.kernel(
        _gather_body,
        mesh=mesh,
        out_type=jax.ShapeDtypeStruct((TOTAL, EMBED_DIM), jnp.float32),
        scratch_types=[
            pltpu.VMEM((IDX_ROWS_PER_GROUP, DMA_CHUNK), jnp.int32),
            pltpu.VMEM((GROUP, EMBED_DIM), jnp.float32),
            pltpu.SemaphoreType.DMA,
        ],
        compiler_params=pltpu.CompilerParams(use_tc_tiling_on_sc=False),
    )
    return run(idx2d, table)


def kernel(item_inputs, table):
    flat = item_inputs.reshape(TOTAL).astype(jnp.int32)
    idx2d = flat.reshape(TOTAL // DMA_CHUNK, DMA_CHUNK)
    out = _sc_gather(idx2d, table)
    return out.reshape(BATCH, HIST, EMBED_DIM)


# speedup vs baseline: 1.1130x; 1.0168x over previous
import functools

import jax
import jax.numpy as jnp
from jax import lax
from jax.experimental import pallas as pl
from jax.experimental.pallas import tpu as pltpu
from jax.experimental.pallas import tpu_sc as plsc

BATCH = 16384
HIST = 50
EMBED_DIM = 32
TOTAL = BATCH * HIST

NUM_CORES = 2
NUM_SUBCORES = 16
NUM_WORKERS = NUM_CORES * NUM_SUBCORES
PER_WORKER = TOTAL // NUM_WORKERS

DMA_CHUNK = 128
GROUP = 1280
DMAS_PER_GROUP = GROUP // DMA_CHUNK
GROUPS_PER_WORKER = PER_WORKER // GROUP
PAIRS = GROUPS_PER_WORKER // 2


def _gather_body(idx_hbm, table_hbm, out_hbm,
                 idx0, idx1, rows0, rows1, sem_g0, sem_g1, sem_s0, sem_s1):
    wid = lax.axis_index("s") * NUM_CORES + lax.axis_index("c")
    base = wid * PER_WORKER

    def load_idx(buf, g):
        gb = pl.multiple_of(base + g * GROUP, 8)
        pltpu.sync_copy(idx_hbm.at[pl.ds(gb, GROUP)], buf)

    def fire_gathers(idx_buf, rows_buf, sem):
        for j in range(DMAS_PER_GROUP):
            sl = pl.ds(j * DMA_CHUNK, DMA_CHUNK)
            pltpu.async_copy(table_hbm.at[idx_buf.at[sl]], rows_buf.at[sl], sem)

    def drain_gathers(idx_buf, rows_buf, sem):
        for j in range(DMAS_PER_GROUP):
            sl = pl.ds(j * DMA_CHUNK, DMA_CHUNK)
            pltpu.make_async_copy(table_hbm.at[idx_buf.at[sl]], rows_buf.at[sl],
                                  sem).wait()

    def fire_store(rows_buf, sem, g):
        gb = pl.multiple_of(base + g * GROUP, 8)
        pltpu.async_copy(rows_buf, out_hbm.at[pl.ds(gb, GROUP)], sem)

    def drain_store(rows_buf, sem, g):
        gb = pl.multiple_of(base + g * GROUP, 8)
        pltpu.make_async_copy(rows_buf, out_hbm.at[pl.ds(gb, GROUP)], sem).wait()

    load_idx(idx0, 0)
    fire_gathers(idx0, rows0, sem_g0)
    load_idx(idx1, 1)
    fire_gathers(idx1, rows1, sem_g1)

    def pair(k, carry):
        g = 2 * k
        drain_gathers(idx0, rows0, sem_g0)
        fire_store(rows0, sem_s0, g)
        load_idx(idx0, g + 2)
        drain_store(rows0, sem_s0, g)
        fire_gathers(idx0, rows0, sem_g0)
        drain_gathers(idx1, rows1, sem_g1)
        fire_store(rows1, sem_s1, g + 1)
        load_idx(idx1, g + 3)
        drain_store(rows1, sem_s1, g + 1)
        fire_gathers(idx1, rows1, sem_g1)
        return carry

    lax.fori_loop(0, PAIRS - 1, pair, 0)

    g_last = GROUPS_PER_WORKER - 2
    drain_gathers(idx0, rows0, sem_g0)
    fire_store(rows0, sem_s0, g_last)
    drain_gathers(idx1, rows1, sem_g1)
    fire_store(rows1, sem_s1, g_last + 1)
    drain_store(rows0, sem_s0, g_last)
    drain_store(rows1, sem_s1, g_last + 1)


@functools.partial(jax.jit, donate_argnums=())
def _sc_gather(idx_flat, table):
    mesh = plsc.VectorSubcoreMesh(core_axis_name="c", subcore_axis_name="s")
    run = pl.kernel(
        _gather_body,
        mesh=mesh,
        out_type=jax.ShapeDtypeStruct((TOTAL, EMBED_DIM), jnp.float32),
        scratch_types=[
            pltpu.VMEM((GROUP,), jnp.int32),
            pltpu.VMEM((GROUP,), jnp.int32),
            pltpu.VMEM((GROUP, EMBED_DIM), jnp.float32),
            pltpu.VMEM((GROUP, EMBED_DIM), jnp.float32),
            pltpu.SemaphoreType.DMA,
            pltpu.SemaphoreType.DMA,
            pltpu.SemaphoreType.DMA,
            pltpu.SemaphoreType.DMA,
        ],
        compiler_params=pltpu.CompilerParams(use_tc_tiling_on_sc=False),
    )
    return run(idx_flat, table)


def kernel(item_inputs, table):
    flat = item_inputs.reshape(TOTAL).astype(jnp.int32)
    out = _sc_gather(flat, table)
    return out.reshape(BATCH, HIST, EMBED_DIM)


# speedup vs baseline: 1.1139x; 1.0008x over previous
import functools

import jax
import jax.numpy as jnp
from jax import lax
from jax.experimental import pallas as pl
from jax.experimental.pallas import tpu as pltpu
from jax.experimental.pallas import tpu_sc as plsc

BATCH = 16384
HIST = 50
EMBED_DIM = 32
TOTAL = BATCH * HIST

NUM_CORES = 2
NUM_SUBCORES = 16
NUM_WORKERS = NUM_CORES * NUM_SUBCORES
PER_WORKER = TOTAL // NUM_WORKERS

DMA_CHUNK = 1280
GROUP = 1280
DMAS_PER_GROUP = GROUP // DMA_CHUNK
GROUPS_PER_WORKER = PER_WORKER // GROUP
PAIRS = GROUPS_PER_WORKER // 2


def _gather_body(idx_hbm, table_hbm, out_hbm,
                 idx0, idx1, rows0, rows1, sem_g0, sem_g1, sem_s0, sem_s1):
    wid = lax.axis_index("s") * NUM_CORES + lax.axis_index("c")
    base = wid * PER_WORKER

    def load_idx(buf, g):
        gb = pl.multiple_of(base + g * GROUP, 8)
        pltpu.sync_copy(idx_hbm.at[pl.ds(gb, GROUP)], buf)

    def fire_gathers(idx_buf, rows_buf, sem):
        for j in range(DMAS_PER_GROUP):
            sl = pl.ds(j * DMA_CHUNK, DMA_CHUNK)
            pltpu.async_copy(table_hbm.at[idx_buf.at[sl]], rows_buf.at[sl], sem)

    def drain_gathers(idx_buf, rows_buf, sem):
        for j in range(DMAS_PER_GROUP):
            sl = pl.ds(j * DMA_CHUNK, DMA_CHUNK)
            pltpu.make_async_copy(table_hbm.at[idx_buf.at[sl]], rows_buf.at[sl],
                                  sem).wait()

    def fire_store(rows_buf, sem, g):
        gb = pl.multiple_of(base + g * GROUP, 8)
        pltpu.async_copy(rows_buf, out_hbm.at[pl.ds(gb, GROUP)], sem)

    def drain_store(rows_buf, sem, g):
        gb = pl.multiple_of(base + g * GROUP, 8)
        pltpu.make_async_copy(rows_buf, out_hbm.at[pl.ds(gb, GROUP)], sem).wait()

    load_idx(idx0, 0)
    fire_gathers(idx0, rows0, sem_g0)
    load_idx(idx1, 1)
    fire_gathers(idx1, rows1, sem_g1)

    def pair(k, carry):
        g = 2 * k
        drain_gathers(idx0, rows0, sem_g0)
        fire_store(rows0, sem_s0, g)
        load_idx(idx0, g + 2)
        drain_store(rows0, sem_s0, g)
        fire_gathers(idx0, rows0, sem_g0)
        drain_gathers(idx1, rows1, sem_g1)
        fire_store(rows1, sem_s1, g + 1)
        load_idx(idx1, g + 3)
        drain_store(rows1, sem_s1, g + 1)
        fire_gathers(idx1, rows1, sem_g1)
        return carry

    lax.fori_loop(0, PAIRS - 1, pair, 0)

    g_last = GROUPS_PER_WORKER - 2
    drain_gathers(idx0, rows0, sem_g0)
    fire_store(rows0, sem_s0, g_last)
    drain_gathers(idx1, rows1, sem_g1)
    fire_store(rows1, sem_s1, g_last + 1)
    drain_store(rows0, sem_s0, g_last)
    drain_store(rows1, sem_s1, g_last + 1)


@functools.partial(jax.jit, donate_argnums=())
def _sc_gather(idx_flat, table):
    mesh = plsc.VectorSubcoreMesh(core_axis_name="c", subcore_axis_name="s")
    run = pl.kernel(
        _gather_body,
        mesh=mesh,
        out_type=jax.ShapeDtypeStruct((TOTAL, EMBED_DIM), jnp.float32),
        scratch_types=[
            pltpu.VMEM((GROUP,), jnp.int32),
            pltpu.VMEM((GROUP,), jnp.int32),
            pltpu.VMEM((GROUP, EMBED_DIM), jnp.float32),
            pltpu.VMEM((GROUP, EMBED_DIM), jnp.float32),
            pltpu.SemaphoreType.DMA,
            pltpu.SemaphoreType.DMA,
            pltpu.SemaphoreType.DMA,
            pltpu.SemaphoreType.DMA,
        ],
        compiler_params=pltpu.CompilerParams(use_tc_tiling_on_sc=False),
    )
    return run(idx_flat, table)


def kernel(item_inputs, table):
    flat = item_inputs.reshape(TOTAL).astype(jnp.int32)
    out = _sc_gather(flat, table)
    return out.reshape(BATCH, HIST, EMBED_DIM)
